# Initial kernel scaffold; baseline (speedup 1.0000x reference)
import jax
import jax.numpy as jnp
from jax.experimental import pallas as pl


def kernel(attn_weights):
    raise NotImplementedError("write your pallas kernel here")


# speedup vs baseline: 236.6894x; 236.6894x over previous
import functools

import jax
import jax.numpy as jnp
from jax.experimental import pallas as pl

_SPARSITY_RATIO = 0.3
_INT32_MIN = -2147483648


def _topk_mask_body(x_ref, o_ref, *, k):
    x = x_ref[...]
    bits = jax.lax.bitcast_convert_type(x, jnp.int32)
    key = bits ^ ((bits >> 31) & jnp.int32(0x7FFFFFFF))

    cnt = jnp.sum((key >= 0).astype(jnp.int32), axis=1, keepdims=True)
    t = jnp.where(cnt >= k, jnp.int32(0), jnp.int32(_INT32_MIN))
    for b in range(30, -1, -1):
        cand = t | jnp.int32(1 << b)
        cnt = jnp.sum((key >= cand).astype(jnp.int32), axis=1, keepdims=True)
        t = jnp.where(cnt >= k, cand, t)

    o_ref[...] = jnp.where(key >= t, x, jnp.float32(0.0))


def kernel(attn_weights):
    shape = attn_weights.shape
    S = shape[-1]
    k = int(S * (1.0 - _SPARSITY_RATIO))
    if k <= 0:
        return attn_weights
    x = attn_weights.reshape(-1, S)
    n = x.shape[0]
    R = 256 if n % 256 == 0 else n
    out = pl.pallas_call(
        functools.partial(_topk_mask_body, k=k),
        grid=(n // R,),
        in_specs=[pl.BlockSpec((R, S), lambda i: (i, 0))],
        out_specs=pl.BlockSpec((R, S), lambda i: (i, 0)),
        out_shape=jax.ShapeDtypeStruct((n, S), x.dtype),
    )(x)
    return out.reshape(shape)
